# Initial kernel scaffold; baseline (speedup 1.0000x reference)
import jax
import jax.numpy as jnp
from jax.experimental import pallas as pl


def kernel(x, edge_index, batch, W1, b1, W2, b2, Wlin, blin):
    raise NotImplementedError("write your pallas kernel here")


# speedup vs baseline: 1.0012x; 1.0012x over previous
import functools

import jax
import jax.numpy as jnp
from jax.experimental import pallas as pl
from jax.experimental.pallas import tpu as pltpu

N = 100000
E = 6400000
NUM_GRAPHS = 64
HIDDEN = 32

POOL_BLK = 2000
POOL_NB = N // POOL_BLK


def _pool_head_kernel(batch_ref, h_ref, wlin_ref, blin_ref, out_ref, acc_sum, acc_cnt):
    i = pl.program_id(0)

    @pl.when(i == 0)
    def _init():
        acc_sum[...] = jnp.zeros_like(acc_sum)
        acc_cnt[...] = jnp.zeros_like(acc_cnt)

    b = batch_ref[0, 0, :]
    h = h_ref[...]
    seg = jax.lax.broadcasted_iota(jnp.int32, (POOL_BLK, NUM_GRAPHS), 1)
    onehot = (b[:, None] == seg).astype(jnp.float32)
    dn = (((0,), (0,)), ((), ()))
    acc_sum[...] += jax.lax.dot_general(
        onehot, h, dn, preferred_element_type=jnp.float32)
    ones = jnp.ones((POOL_BLK, 1), dtype=jnp.float32)
    acc_cnt[...] += jax.lax.dot_general(
        onehot, ones, dn, preferred_element_type=jnp.float32)

    @pl.when(i == POOL_NB - 1)
    def _emit():
        pooled = acc_sum[...] / jnp.maximum(acc_cnt[...], 1.0)
        out_ref[...] = (
            jnp.dot(pooled, wlin_ref[...], preferred_element_type=jnp.float32)
            + blin_ref[...]
        )


def _pool_head(h2, batch, Wlin, blin):
    batch3 = batch.reshape(POOL_NB, 1, POOL_BLK)
    out = pl.pallas_call(
        _pool_head_kernel,
        grid=(POOL_NB,),
        in_specs=[
            pl.BlockSpec((1, 1, POOL_BLK), lambda i: (i, 0, 0)),
            pl.BlockSpec((POOL_BLK, HIDDEN), lambda i: (i, 0)),
            pl.BlockSpec((HIDDEN, 1), lambda i: (0, 0)),
            pl.BlockSpec((1, 1), lambda i: (0, 0)),
        ],
        out_specs=pl.BlockSpec((NUM_GRAPHS, 1), lambda i: (0, 0)),
        out_shape=jax.ShapeDtypeStruct((NUM_GRAPHS, 1), jnp.float32),
        scratch_shapes=[
            pltpu.VMEM((NUM_GRAPHS, HIDDEN), jnp.float32),
            pltpu.VMEM((NUM_GRAPHS, 1), jnp.float32),
        ],
    )(batch3, h2, Wlin, blin.reshape(1, 1))
    return out[:, 0]


def _gcn_conv(x, edge_index, W, b):
    src = edge_index[0]
    dst = edge_index[1]
    loop = jnp.arange(N, dtype=src.dtype)
    src = jnp.concatenate([src, loop])
    dst = jnp.concatenate([dst, loop])
    deg = jnp.zeros((N,), dtype=jnp.float32).at[dst].add(1.0)
    deg_inv_sqrt = jnp.where(deg > 0, jax.lax.rsqrt(jnp.maximum(deg, 1e-12)), 0.0)
    norm = deg_inv_sqrt[src] * deg_inv_sqrt[dst]
    h = x @ W
    msg = h[src] * norm[:, None]
    out = jnp.zeros((N, W.shape[1]), dtype=jnp.float32).at[dst].add(msg)
    return out + b


def kernel(x, edge_index, batch, W1, b1, W2, b2, Wlin, blin):
    h = jax.nn.relu(_gcn_conv(x, edge_index, W1, b1))
    h = jax.nn.relu(_gcn_conv(h, edge_index, W2, b2))
    return _pool_head(h, batch, Wlin, blin)


# speedup vs baseline: 30.2041x; 30.1673x over previous
import functools

import jax
import jax.numpy as jnp
from jax import lax
from jax.experimental import pallas as pl
from jax.experimental.pallas import tpu as pltpu
from jax.experimental.pallas import tpu_sc as plsc

N = 100000
E = 6400000
G = 64
H = 32
C = 16

NC, NS = 2, 16
NW = NC * NS

NPAD = 102400
HALF = NPAD // 2
RPS = HALF // NS
SROWS = 1600
EB = 2000
EPW = E // NW
EPS = E // NS

BLK2 = 4096
NB2 = NPAD // BLK2
PBLK = 2048
PNB = NPAD // PBLK


def _sc_mesh():
    return plsc.VectorSubcoreMesh(core_axis_name="c", subcore_axis_name="s")


def _deg_pallas(dst, ones_eb, zeros_1d):
    @functools.partial(
        pl.kernel,
        out_type=jax.ShapeDtypeStruct((NW, NPAD // NS), jnp.float32),
        mesh=_sc_mesh(),
        compiler_params=pltpu.CompilerParams(use_tc_tiling_on_sc=False),
        scratch_types=[
            pltpu.VMEM((EB,), jnp.int32),
            pltpu.VMEM((EB,), jnp.float32),
            pltpu.VMEM((NPAD // NS,), jnp.float32),
            pltpu.VMEM_SHARED((NPAD,), jnp.float32),
        ],
    )
    def body(dst_hbm, ones_hbm, zeros_hbm, out_hbm, didx_v, ones_v, stage_v, acc_sh):
        c = lax.axis_index("c")
        s = lax.axis_index("s")
        w = s * NC + c
        zrows = NPAD // NS
        r0 = s * zrows
        pltpu.sync_copy(zeros_hbm, stage_v)
        pltpu.sync_copy(stage_v, acc_sh.at[pl.ds(r0, zrows)])
        plsc.subcore_barrier()
        pltpu.sync_copy(ones_hbm, ones_v)

        def step(i, carry):
            base = w * EPW + i * EB
            pltpu.sync_copy(dst_hbm.at[pl.ds(base, EB)], didx_v)
            pltpu.sync_copy(ones_v, acc_sh.at[didx_v], add=True)
            return carry

        lax.fori_loop(0, EPW // EB, step, 0)
        plsc.subcore_barrier()
        pltpu.sync_copy(acc_sh.at[pl.ds(r0, zrows)], stage_v)
        pltpu.sync_copy(stage_v, out_hbm.at[c * NS + s])

    return body(dst, ones_eb, zeros_1d)


def _filter_chunk(sidx_v, didx_v, fsidx_v, fdidx_v, lo, hi):
    def fstep(k, carry):
        o = pl.multiple_of(k * 16, 16)
        sl = sidx_v[pl.ds(o, 16)]
        dl = didx_v[pl.ds(o, 16)]
        m = (dl >= lo) & (dl < hi)
        fsidx_v[pl.ds(o, 16)] = jnp.where(m, sl, -1)
        fdidx_v[pl.ds(o, 16)] = jnp.where(m, dl - lo, -1)
        return carry

    lax.fori_loop(0, EB // 16, fstep, 0)


def _scatter_rows(tables):
    nj = RPS // SROWS
    out_ty = jax.ShapeDtypeStruct(
        (tables * NC * NS * nj, SROWS, C), jnp.float32)

    @functools.partial(
        pl.kernel,
        out_type=out_ty,
        mesh=_sc_mesh(),
        compiler_params=pltpu.CompilerParams(use_tc_tiling_on_sc=False),
        scratch_types=[
            pltpu.VMEM((EB,), jnp.int32),
            pltpu.VMEM((EB,), jnp.int32),
            pltpu.VMEM((EB,), jnp.int32),
            pltpu.VMEM((EB,), jnp.int32),
            pltpu.VMEM((EB, C), jnp.float32),
            pltpu.VMEM((SROWS, C), jnp.float32),
            pltpu.VMEM_SHARED((HALF, C), jnp.float32),
        ],
    )
    def body(src_hbm, dst_hbm, y_hbm, zeros_hbm, out_hbm,
             sidx_v, didx_v, fsidx_v, fdidx_v, rows_v, stage_v, acc_sh):
        c = lax.axis_index("c")
        s = lax.axis_index("s")
        lo = c * HALF
        hi = lo + HALF
        pltpu.sync_copy(zeros_hbm, stage_v)

        for t in range(tables):
            table = y_hbm if tables == 1 else y_hbm.at[t]
            for j in range(RPS // SROWS):
                pltpu.sync_copy(
                    stage_v, acc_sh.at[pl.ds(s * RPS + j * SROWS, SROWS)])
            plsc.subcore_barrier()

            def step(i, carry):
                base = s * EPS + i * EB
                pltpu.sync_copy(src_hbm.at[pl.ds(base, EB)], sidx_v)
                pltpu.sync_copy(dst_hbm.at[pl.ds(base, EB)], didx_v)
                _filter_chunk(sidx_v, didx_v, fsidx_v, fdidx_v, lo, hi)
                pltpu.sync_copy(
                    table.at[plsc.Indices(fsidx_v, ignored_value=-1)], rows_v)
                pltpu.sync_copy(
                    rows_v, acc_sh.at[plsc.Indices(fdidx_v, ignored_value=-1)],
                    add=True)
                return carry

            lax.fori_loop(0, EPS // EB, step, 0)
            plsc.subcore_barrier()
            for j in range(RPS // SROWS):
                r0 = s * RPS + j * SROWS
                pltpu.sync_copy(acc_sh.at[pl.ds(r0, SROWS)], stage_v)
                wo = ((t * NC + c) * NS + s) * (RPS // SROWS) + j
                pltpu.sync_copy(stage_v, out_hbm.at[wo])
            if t + 1 < tables:
                pltpu.sync_copy(zeros_hbm, stage_v)

    return body


def _mid0_kernel(degp_ref, x_ref, w1_ref, dis_ref, y1ch_ref):
    deg = degp_ref[0, :] + degp_ref[1, :] + 1.0
    r = lax.rsqrt(deg)
    r = r * (1.5 - 0.5 * deg * r * r)
    dis = r[:, None]
    dis_ref[...] = dis
    h1p = jnp.dot(x_ref[...], w1_ref[...], preferred_element_type=jnp.float32)
    y1 = dis * h1p
    y1ch_ref[0] = y1[:, :C]
    y1ch_ref[1] = y1[:, C:]


def _mid0(degp, x_pad, W1p):
    return pl.pallas_call(
        _mid0_kernel,
        grid=(NB2,),
        in_specs=[
            pl.BlockSpec((2, BLK2), lambda i: (0, i)),
            pl.BlockSpec((BLK2, C), lambda i: (i, 0)),
            pl.BlockSpec((C, H), lambda i: (0, 0)),
        ],
        out_specs=[
            pl.BlockSpec((BLK2, 1), lambda i: (i, 0)),
            pl.BlockSpec((2, BLK2, C), lambda i: (0, i, 0)),
        ],
        out_shape=[
            jax.ShapeDtypeStruct((NPAD, 1), jnp.float32),
            jax.ShapeDtypeStruct((NC, NPAD, C), jnp.float32),
        ],
    )(degp, x_pad, W1p)


def _mid1_kernel(s1ch_ref, y1ch_ref, dis_ref, w2_ref, b1_ref, y2ch_ref):
    m0 = s1ch_ref[0] + y1ch_ref[0]
    m1 = s1ch_ref[1] + y1ch_ref[1]
    t = jnp.concatenate([m0, m1], axis=1) * dis_ref[...]
    h1 = jnp.maximum(t + b1_ref[...], 0.0)
    h2p = jnp.dot(h1, w2_ref[...], preferred_element_type=jnp.float32)
    y2 = dis_ref[...] * h2p
    y2ch_ref[0] = y2[:, :C]
    y2ch_ref[1] = y2[:, C:]


def _mid1(s1ch, y1ch, dis, W2, b1):
    return pl.pallas_call(
        _mid1_kernel,
        grid=(NB2,),
        in_specs=[
            pl.BlockSpec((2, BLK2, C), lambda i: (0, i, 0)),
            pl.BlockSpec((2, BLK2, C), lambda i: (0, i, 0)),
            pl.BlockSpec((BLK2, 1), lambda i: (i, 0)),
            pl.BlockSpec((H, H), lambda i: (0, 0)),
            pl.BlockSpec((1, H), lambda i: (0, 0)),
        ],
        out_specs=pl.BlockSpec((2, BLK2, C), lambda i: (0, i, 0)),
        out_shape=jax.ShapeDtypeStruct((NC, NPAD, C), jnp.float32),
    )(s1ch, y1ch, dis, W2, b1.reshape(1, H))


def _final_kernel(s2ch_ref, y2ch_ref, dis_ref, b2_ref, batch_ref,
                  wlin_ref, blin_ref, out_ref, acc_sum, acc_cnt):
    i = pl.program_id(0)

    @pl.when(i == 0)
    def _init():
        acc_sum[...] = jnp.zeros_like(acc_sum)
        acc_cnt[...] = jnp.zeros_like(acc_cnt)

    m0 = s2ch_ref[0] + y2ch_ref[0]
    m1 = s2ch_ref[1] + y2ch_ref[1]
    t2 = jnp.concatenate([m0, m1], axis=1) * dis_ref[...]
    h2 = jnp.maximum(t2 + b2_ref[...], 0.0)

    b = batch_ref[...]
    seg = lax.broadcasted_iota(jnp.int32, (PBLK, G), 1)
    onehot = (b == seg).astype(jnp.float32)
    dn = (((0,), (0,)), ((), ()))
    acc_sum[...] += lax.dot_general(onehot, h2, dn,
                                    preferred_element_type=jnp.float32,
                                    precision=lax.Precision.HIGHEST)
    ones = jnp.ones((PBLK, 1), dtype=jnp.float32)
    acc_cnt[...] += lax.dot_general(onehot, ones, dn,
                                    preferred_element_type=jnp.float32,
                                    precision=lax.Precision.HIGHEST)

    @pl.when(i == PNB - 1)
    def _emit():
        pooled = acc_sum[...] / jnp.maximum(acc_cnt[...], 1.0)
        out_ref[...] = (
            jnp.dot(pooled, wlin_ref[...], preferred_element_type=jnp.float32)
            + blin_ref[...]
        )


def _final(s2ch, y2ch, dis, b2, batch3, Wlin, blin):
    out = pl.pallas_call(
        _final_kernel,
        grid=(PNB,),
        in_specs=[
            pl.BlockSpec((2, PBLK, C), lambda i: (0, i, 0)),
            pl.BlockSpec((2, PBLK, C), lambda i: (0, i, 0)),
            pl.BlockSpec((PBLK, 1), lambda i: (i, 0)),
            pl.BlockSpec((1, H), lambda i: (0, 0)),
            pl.BlockSpec((PBLK, 1), lambda i: (i, 0)),
            pl.BlockSpec((H, 1), lambda i: (0, 0)),
            pl.BlockSpec((1, 1), lambda i: (0, 0)),
        ],
        out_specs=pl.BlockSpec((G, 1), lambda i: (0, 0)),
        out_shape=jax.ShapeDtypeStruct((G, 1), jnp.float32),
        scratch_shapes=[
            pltpu.VMEM((G, H), jnp.float32),
            pltpu.VMEM((G, 1), jnp.float32),
        ],
    )(s2ch, y2ch, dis, b2.reshape(1, H), batch3, Wlin, blin.reshape(1, 1))
    return out[:, 0]


def kernel(x, edge_index, batch, W1, b1, W2, b2, Wlin, blin):
    src = edge_index[0]
    dst = edge_index[1]
    x_pad = jnp.pad(x, ((0, NPAD - N), (0, C - x.shape[1])))
    W1p = jnp.pad(W1, ((0, C - W1.shape[0]), (0, 0)))
    batch3 = jnp.pad(batch, (0, NPAD - N), constant_values=G).reshape(
        NPAD, 1)
    ones_eb = jnp.ones((EB,), dtype=jnp.float32)
    zeros_1d = jnp.zeros((NPAD // NS,), dtype=jnp.float32)
    zeros_2d = jnp.zeros((SROWS, C), dtype=jnp.float32)

    degj = jnp.zeros((NPAD,), jnp.float32).at[dst].add(1.0)
    degp = _deg_pallas(dst, ones_eb, zeros_1d).reshape(NC, NPAD)
    dis, y1ch = _mid0(degp, x_pad, W1p)
    s1ch = _scatter_rows(tables=2)(src, dst, y1ch, zeros_2d).reshape(
        NC, NPAD, C)
    y2ch = _mid1(s1ch, y1ch, dis, W2, b1)
    s2ch = _scatter_rows(tables=2)(src, dst, y2ch, zeros_2d).reshape(
        NC, NPAD, C)
    return _final(s2ch, y2ch, dis, b2, batch3, Wlin, blin)
